# baseline (speedup 1.0000x reference)
import functools

import jax
import jax.numpy as jnp
from jax import lax
from jax.experimental import pallas as pl
from jax.experimental.pallas import tpu as pltpu
from jax.experimental.pallas import tpu_sc as plsc

F32 = jnp.float32

NC = 2
NS = 16
CHUNK = 128
NBUF = 4


def _round_up(a, b):
    return (a + b - 1) // b * b


def _sc_segsum(src_flat, dst_flat, ztab, tables, n_nodes, split):
    C = len(tables)
    n_chunks = src_flat.shape[0] // CHUNK
    cpw0, cpw1 = split
    assert NS * (cpw0 + cpw1) == n_chunks
    IB = 28
    GRP = CHUNK
    NG = IB * CHUNK // GRP
    NB0, NB1 = cpw0 // IB, cpw1 // IB
    NP = _round_up(n_nodes + 1, 512)
    TPS = NP // NS
    mesh = plsc.VectorSubcoreMesh(core_axis_name="c", subcore_axis_name="s")

    @functools.partial(
        pl.kernel,
        out_type=jax.ShapeDtypeStruct((NC * C * NP, 32), F32),
        mesh=mesh,
        compiler_params=pltpu.CompilerParams(use_tc_tiling_on_sc=False),
        scratch_types=dict(
            acc=pltpu.VMEM_SHARED((NP, 32), F32),
            sidx=pltpu.VMEM((IB * CHUNK,), jnp.int32),
            didx=pltpu.VMEM((IB * CHUNK,), jnp.int32),
            rowbufs=[pltpu.VMEM((GRP, 32), F32) for _ in range(NBUF)],
            gsems=[pltpu.SemaphoreType.DMA for _ in range(NBUF)],
            ssems=[pltpu.SemaphoreType.DMA for _ in range(NBUF)],
        ),
    )
    def ksc(src_hbm, dst_hbm, z_hbm, *args, acc, sidx, didx,
            rowbufs, gsems, ssems):
        tabs = args[:C]
        out_hbm = args[C]
        core = lax.axis_index("c")
        sub = lax.axis_index("s")
        start = jnp.where(core == 0, sub * cpw0, NS * cpw0 + sub * cpw1)
        nb = jnp.where(core == 0, NB0, NB1)

        for c in range(C):
            table = tabs[c]
            pltpu.sync_copy(z_hbm, acc.at[pl.ds(sub * TPS, TPS)])
            plsc.subcore_barrier()

            @pl.loop(0, nb)
            def _blocks(blk):
                ib_off = (start + blk * IB) * CHUNK
                pltpu.sync_copy(
                    src_hbm.at[pl.ds(ib_off, IB * CHUNK)], sidx)
                pltpu.sync_copy(
                    dst_hbm.at[pl.ds(ib_off, IB * CHUNK)], didx)
                for b in range(NBUF):
                    pltpu.async_copy(
                        table.at[sidx.at[pl.ds(b * GRP, GRP)]],
                        rowbufs[b], gsems[b])

                @pl.loop(0, NG, step=NBUF)
                def _chunks(i0):
                    for b in range(NBUF):
                        j = i0 + b
                        pltpu.make_async_copy(
                            table.at[sidx.at[pl.ds(j * GRP, GRP)]],
                            rowbufs[b], gsems[b]).wait()
                        pltpu.async_copy(
                            rowbufs[b],
                            acc.at[didx.at[pl.ds(j * GRP, GRP)]],
                            ssems[b], add=True)

                        @pl.when(j + NBUF < NG)
                        def _():
                            pltpu.make_async_copy(
                                rowbufs[b],
                                acc.at[didx.at[pl.ds(j * GRP, GRP)]],
                                ssems[b]).wait()
                            pltpu.async_copy(
                                table.at[sidx.at[pl.ds((j + NBUF) * GRP, GRP)]],
                                rowbufs[b], gsems[b])

                for b in range(NBUF):
                    j = NG - NBUF + b
                    pltpu.make_async_copy(
                        rowbufs[j % NBUF],
                        acc.at[didx.at[pl.ds(j * GRP, GRP)]],
                        ssems[j % NBUF]).wait()

            plsc.subcore_barrier()
            base = (core * C + c) * NP + sub * TPS
            pltpu.sync_copy(acc.at[pl.ds(sub * TPS, TPS)],
                            out_hbm.at[pl.ds(base, TPS)])

    out = ksc(src_flat, dst_flat, ztab, *tables)
    return out.reshape(NC, C, NP, 32)


def _tc_layer1(x, SEL, W1bd, b1p):
    n, k = x.shape
    bn = 512
    ng = _round_up(n, bn) // bn
    rp = ng * bn // 4

    def body(x_ref, s_ref, w_ref, b_ref, *out_refs):
        xc = jnp.clip(x_ref[...], -1.8, 1.8)
        xq = jnp.concatenate(
            [jnp.dot(s_ref[a], xc, preferred_element_type=F32)
             for a in range(4)], axis=1)
        h = jnp.dot(xq, w_ref[...], preferred_element_type=F32) + b_ref[...]
        for c in range(4):
            out_refs[c][...] = h[:, 128 * c:128 * (c + 1)]

    return pl.pallas_call(
        body,
        grid=(ng,),
        in_specs=[
            pl.BlockSpec((bn, k), lambda i: (i, 0)),
            pl.BlockSpec((4, bn // 4, bn), lambda i: (0, 0, 0)),
            pl.BlockSpec((4 * k, 512), lambda i: (0, 0)),
            pl.BlockSpec((1, 512), lambda i: (0, 0)),
        ],
        out_specs=[pl.BlockSpec((bn // 4, 128), lambda i: (i, 0))] * 4,
        out_shape=[jax.ShapeDtypeStruct((rp, 128), F32)] * 4,
    )(x, SEL, W1bd, b1p.reshape(1, 512))


def _tc_mid(p, Wbd, bp, c_out):
    rp = p.shape[2]
    bn = 128
    grid = (rp // bn,)
    cols = 128 * c_out

    def body(p_ref, w_ref, b_ref, *out_refs):
        h = jnp.concatenate(
            [jnp.maximum(p_ref[0, c] + p_ref[1, c], 0.0) for c in range(4)],
            axis=1)
        r = jnp.dot(h, w_ref[...], preferred_element_type=F32) + b_ref[...]
        for c in range(c_out):
            out_refs[c][...] = r[:, 128 * c:128 * (c + 1)]

    return pl.pallas_call(
        body,
        grid=grid,
        in_specs=[
            pl.BlockSpec((2, 4, bn, 128), lambda i: (0, 0, i, 0)),
            pl.BlockSpec((512, cols), lambda i: (0, 0)),
            pl.BlockSpec((1, cols), lambda i: (0, 0)),
        ],
        out_specs=[pl.BlockSpec((bn, 128), lambda i: (i, 0))] * c_out,
        out_shape=[jax.ShapeDtypeStruct((rp, 128), F32)] * c_out,
    )(p, Wbd, bp.reshape(1, -1))


def _tc_final(p):
    rp = p.shape[2]
    bn = 128
    grid = (rp // bn,)

    def body(p_ref, out_ref):
        out_ref[...] = p_ref[0, 0] + p_ref[1, 0]

    return pl.pallas_call(
        body,
        grid=grid,
        in_specs=[pl.BlockSpec((2, 1, bn, 128), lambda i: (0, 0, i, 0))],
        out_specs=pl.BlockSpec((bn, 128), lambda i: (i, 0)),
        out_shape=jax.ShapeDtypeStruct((rp, 128), F32),
    )(p)


def kernel(x, adj, W1, b1, W2, b2, W3, b3):
    n = x.shape[0]
    e = adj.shape[1]
    NP = _round_up(n + 1, 512)

    cpw = -(-e // (CHUNK * NC * NS))
    ep = cpw * CHUNK * NC * NS
    src_flat = jnp.concatenate(
        [adj[0], jnp.zeros((ep - e,), jnp.int32)])
    dst_flat = jnp.concatenate(
        [adj[1], jnp.full((ep - e,), n, jnp.int32)])
    split = (168, 224)
    assert NS * sum(split) == ep // CHUNK
    ztab = jnp.zeros((NP // NS, 32), F32)

    W1r = W1.reshape(W1.shape[0], 4, 32)
    K = W1.shape[0]
    W1bd = (jnp.eye(4, dtype=F32)[:, None, None, :, None]
            * W1r[None, :, :, None, :]).reshape(4 * K, 512)
    b1p = jnp.tile(b1.reshape(4, 1, 32), (1, 4, 1)).reshape(512)

    W2r = W2.reshape(4, 32, 4, 32)
    W2bd = (jnp.eye(4, dtype=F32)[None, :, None, None, :, None]
            * W2r[:, None, :, :, None, :]).reshape(512, 512)
    b2p = jnp.tile(b2.reshape(4, 1, 32), (1, 4, 1)).reshape(512)

    W3p = jnp.pad(W3, ((0, 0), (0, 32 - W3.shape[1])))
    W3r = W3p.reshape(4, 32, 32)
    W3bd = (jnp.eye(4, dtype=F32)[None, :, None, :, None]
            * W3r[:, None, :, None, :]).reshape(512, 128)
    b3p = jnp.tile(jnp.pad(b3, (0, 32 - b3.shape[0])).reshape(1, 32),
                   (4, 1)).reshape(128)

    SEL = jnp.eye(512, dtype=F32).reshape(128, 4, 512).transpose(1, 0, 2)

    g1 = _tc_layer1(x, SEL, W1bd, b1p)
    p1 = _sc_segsum(src_flat, dst_flat, ztab,
                    [t.reshape(-1, 32) for t in g1], n, split)
    g2 = _tc_mid(p1.reshape(NC, 4, NP // 4, 128), W2bd, b2p, 4)
    p2 = _sc_segsum(src_flat, dst_flat, ztab,
                    [t.reshape(-1, 32) for t in g2], n, split)
    g3 = _tc_mid(p2.reshape(NC, 4, NP // 4, 128), W3bd, b3p, 1)
    p3 = _sc_segsum(src_flat, dst_flat, ztab,
                    [t.reshape(-1, 32) for t in g3], n, split)
    out = _tc_final(p3.reshape(NC, 1, NP // 4, 128))
    return out.reshape(NP, 32)[:n, :18]


# speedup vs baseline: 1.1142x; 1.1142x over previous
import functools

import jax
import jax.numpy as jnp
from jax import lax
from jax.experimental import pallas as pl
from jax.experimental.pallas import tpu as pltpu
from jax.experimental.pallas import tpu_sc as plsc

F32 = jnp.float32

NC = 2
NS = 16
CHUNK = 128
NBUF = 4


def _round_up(a, b):
    return (a + b - 1) // b * b


def _sc_segsum(src_flat, dst_flat, ztab, tables, n_nodes, split):
    C = len(tables)
    n_chunks = src_flat.shape[0] // CHUNK
    cpw0, cpw1 = split
    assert NS * (cpw0 + cpw1) == n_chunks
    IB = 28
    GRP = CHUNK
    NG = IB * CHUNK // GRP
    NB0, NB1 = cpw0 // IB, cpw1 // IB
    NP = _round_up(n_nodes + 1, 512)
    TPS = NP // NS
    mesh = plsc.VectorSubcoreMesh(core_axis_name="c", subcore_axis_name="s")

    @functools.partial(
        pl.kernel,
        out_type=jax.ShapeDtypeStruct((NC * C * NP, 32), F32),
        mesh=mesh,
        compiler_params=pltpu.CompilerParams(use_tc_tiling_on_sc=False),
        scratch_types=dict(
            acc=pltpu.VMEM_SHARED((NP, 32), F32),
            sidx=pltpu.VMEM((IB * CHUNK,), jnp.int32),
            didx=pltpu.VMEM((IB * CHUNK,), jnp.int32),
            rowbufs=[pltpu.VMEM((GRP, 32), F32) for _ in range(NBUF)],
            gsems=[pltpu.SemaphoreType.DMA for _ in range(NBUF)],
            ssems=[pltpu.SemaphoreType.DMA for _ in range(NBUF)],
        ),
    )
    def ksc(src_hbm, dst_hbm, z_hbm, *args, acc, sidx, didx,
            rowbufs, gsems, ssems):
        tabs = args[:C]
        out_hbm = args[C]
        core = lax.axis_index("c")
        sub = lax.axis_index("s")
        start = jnp.where(core == 0, sub * cpw0, NS * cpw0 + sub * cpw1)
        nb = jnp.where(core == 0, NB0, NB1)

        for c in range(C):
            table = tabs[c]
            pltpu.sync_copy(z_hbm, acc.at[pl.ds(sub * TPS, TPS)])
            plsc.subcore_barrier()

            @pl.loop(0, nb)
            def _blocks(blk):
                ib_off = (start + blk * IB) * CHUNK
                pltpu.sync_copy(
                    src_hbm.at[pl.ds(ib_off, IB * CHUNK)], sidx)
                pltpu.sync_copy(
                    dst_hbm.at[pl.ds(ib_off, IB * CHUNK)], didx)
                for b in range(NBUF):
                    pltpu.async_copy(
                        table.at[sidx.at[pl.ds(b * GRP, GRP)]],
                        rowbufs[b], gsems[b])

                @pl.loop(0, NG, step=NBUF)
                def _chunks(i0):
                    for b in range(NBUF):
                        j = i0 + b
                        pltpu.make_async_copy(
                            table.at[sidx.at[pl.ds(j * GRP, GRP)]],
                            rowbufs[b], gsems[b]).wait()
                        pltpu.async_copy(
                            rowbufs[b],
                            acc.at[didx.at[pl.ds(j * GRP, GRP)]],
                            ssems[b], add=True)

                        @pl.when(j + NBUF < NG)
                        def _():
                            pltpu.make_async_copy(
                                rowbufs[b],
                                acc.at[didx.at[pl.ds(j * GRP, GRP)]],
                                ssems[b]).wait()
                            pltpu.async_copy(
                                table.at[sidx.at[pl.ds((j + NBUF) * GRP, GRP)]],
                                rowbufs[b], gsems[b])

                for b in range(NBUF):
                    j = NG - NBUF + b
                    pltpu.make_async_copy(
                        rowbufs[j % NBUF],
                        acc.at[didx.at[pl.ds(j * GRP, GRP)]],
                        ssems[j % NBUF]).wait()

            plsc.subcore_barrier()
            base = (core * C + c) * NP + sub * TPS
            pltpu.sync_copy(acc.at[pl.ds(sub * TPS, TPS)],
                            out_hbm.at[pl.ds(base, TPS)])

    out = ksc(src_flat, dst_flat, ztab, *tables)
    return out.reshape(NC, C, NP, 32)


def _tc_layer1(x, SEL, W1bd, b1p):
    n, k = x.shape
    bn = 512
    ng = _round_up(n, bn) // bn
    rp = ng * bn // 4

    def body(x_ref, s_ref, w_ref, b_ref, *out_refs):
        xc = jnp.clip(x_ref[...], -1.8, 1.8)
        xq = jnp.concatenate(
            [jnp.dot(s_ref[a], xc, preferred_element_type=F32)
             for a in range(4)], axis=1)
        h = jnp.dot(xq, w_ref[...], preferred_element_type=F32) + b_ref[...]
        for c in range(4):
            out_refs[c][...] = h[:, 128 * c:128 * (c + 1)]

    return pl.pallas_call(
        body,
        grid=(ng,),
        in_specs=[
            pl.BlockSpec((bn, k), lambda i: (i, 0)),
            pl.BlockSpec((4, bn // 4, bn), lambda i: (0, 0, 0)),
            pl.BlockSpec((4 * k, 512), lambda i: (0, 0)),
            pl.BlockSpec((1, 512), lambda i: (0, 0)),
        ],
        out_specs=[pl.BlockSpec((bn // 4, 128), lambda i: (i, 0))] * 4,
        out_shape=[jax.ShapeDtypeStruct((rp, 128), F32)] * 4,
    )(x, SEL, W1bd, b1p.reshape(1, 512))


def _tc_mid(p, Wbd, bp, c_out):
    rp = p.shape[2]
    bn = 128
    grid = (rp // bn,)
    cols = 128 * c_out

    def body(p_ref, w_ref, b_ref, *out_refs):
        h = jnp.concatenate(
            [jnp.maximum(p_ref[0, c] + p_ref[1, c], 0.0) for c in range(4)],
            axis=1)
        r = jnp.dot(h, w_ref[...], preferred_element_type=F32) + b_ref[...]
        for c in range(c_out):
            out_refs[c][...] = r[:, 128 * c:128 * (c + 1)]

    return pl.pallas_call(
        body,
        grid=grid,
        in_specs=[
            pl.BlockSpec((2, 4, bn, 128), lambda i: (0, 0, i, 0)),
            pl.BlockSpec((512, cols), lambda i: (0, 0)),
            pl.BlockSpec((1, cols), lambda i: (0, 0)),
        ],
        out_specs=[pl.BlockSpec((bn, 128), lambda i: (i, 0))] * c_out,
        out_shape=[jax.ShapeDtypeStruct((rp, 128), F32)] * c_out,
    )(p, Wbd, bp.reshape(1, -1))


def _tc_final(p):
    rp = p.shape[2]
    bn = 128
    grid = (rp // bn,)

    def body(p_ref, out_ref):
        out_ref[...] = p_ref[0, 0] + p_ref[1, 0]

    return pl.pallas_call(
        body,
        grid=grid,
        in_specs=[pl.BlockSpec((2, 1, bn, 128), lambda i: (0, 0, i, 0))],
        out_specs=pl.BlockSpec((bn, 128), lambda i: (i, 0)),
        out_shape=jax.ShapeDtypeStruct((rp, 128), F32),
    )(p)


def kernel(x, adj, W1, b1, W2, b2, W3, b3):
    n = x.shape[0]
    e = adj.shape[1]
    NP = _round_up(n + 1, 512)

    cpw = -(-e // (CHUNK * NC * NS))
    ep = cpw * CHUNK * NC * NS
    src_flat = jnp.concatenate(
        [adj[0], jnp.zeros((ep - e,), jnp.int32)])
    dst_flat = jnp.concatenate(
        [adj[1], jnp.full((ep - e,), n, jnp.int32)])
    split = (224, 168)
    assert NS * sum(split) == ep // CHUNK
    ztab = jnp.zeros((NP // NS, 32), F32)

    W1r = W1.reshape(W1.shape[0], 4, 32)
    K = W1.shape[0]
    W1bd = (jnp.eye(4, dtype=F32)[:, None, None, :, None]
            * W1r[None, :, :, None, :]).reshape(4 * K, 512)
    b1p = jnp.tile(b1.reshape(4, 1, 32), (1, 4, 1)).reshape(512)

    W2r = W2.reshape(4, 32, 4, 32)
    W2bd = (jnp.eye(4, dtype=F32)[None, :, None, None, :, None]
            * W2r[:, None, :, :, None, :]).reshape(512, 512)
    b2p = jnp.tile(b2.reshape(4, 1, 32), (1, 4, 1)).reshape(512)

    W3p = jnp.pad(W3, ((0, 0), (0, 32 - W3.shape[1])))
    W3r = W3p.reshape(4, 32, 32)
    W3bd = (jnp.eye(4, dtype=F32)[None, :, None, :, None]
            * W3r[:, None, :, None, :]).reshape(512, 128)
    b3p = jnp.tile(jnp.pad(b3, (0, 32 - b3.shape[0])).reshape(1, 32),
                   (4, 1)).reshape(128)

    SEL = jnp.eye(512, dtype=F32).reshape(128, 4, 512).transpose(1, 0, 2)

    g1 = _tc_layer1(x, SEL, W1bd, b1p)
    p1 = _sc_segsum(src_flat, dst_flat, ztab,
                    [t.reshape(-1, 32) for t in g1], n, split)
    g2 = _tc_mid(p1.reshape(NC, 4, NP // 4, 128), W2bd, b2p, 4)
    p2 = _sc_segsum(src_flat, dst_flat, ztab,
                    [t.reshape(-1, 32) for t in g2], n, split)
    g3 = _tc_mid(p2.reshape(NC, 4, NP // 4, 128), W3bd, b3p, 1)
    p3 = _sc_segsum(src_flat, dst_flat, ztab,
                    [t.reshape(-1, 32) for t in g3], n, split)
    out = _tc_final(p3.reshape(NC, 1, NP // 4, 128))
    return out.reshape(NP, 32)[:n, :18]


# speedup vs baseline: 1.1329x; 1.0167x over previous
import functools

import jax
import jax.numpy as jnp
from jax import lax
from jax.experimental import pallas as pl
from jax.experimental.pallas import tpu as pltpu
from jax.experimental.pallas import tpu_sc as plsc

F32 = jnp.float32

NC = 2
NS = 16
CHUNK = 128
NBUF = 4


def _round_up(a, b):
    return (a + b - 1) // b * b


def _sc_segsum(src_flat, dst_flat, ztab, tables, n_nodes, split):
    C = len(tables)
    n_chunks = src_flat.shape[0] // CHUNK
    cpw0, cpw1 = split
    assert NS * (cpw0 + cpw1) == n_chunks
    IB = 28
    GRP = CHUNK
    NG = IB * CHUNK // GRP
    NB0, NB1 = cpw0 // IB, cpw1 // IB
    NP = _round_up(n_nodes + 1, 512)
    TPS = NP // NS
    mesh = plsc.VectorSubcoreMesh(core_axis_name="c", subcore_axis_name="s")

    @functools.partial(
        pl.kernel,
        out_type=jax.ShapeDtypeStruct((NC * C * NP, 32), F32),
        mesh=mesh,
        compiler_params=pltpu.CompilerParams(use_tc_tiling_on_sc=False),
        scratch_types=dict(
            acc=pltpu.VMEM_SHARED((NP, 32), F32),
            sidx=pltpu.VMEM((IB * CHUNK,), jnp.int32),
            didx=pltpu.VMEM((IB * CHUNK,), jnp.int32),
            rowbufs=[pltpu.VMEM((GRP, 32), F32) for _ in range(NBUF)],
            gsems=[pltpu.SemaphoreType.DMA for _ in range(NBUF)],
            ssems=[pltpu.SemaphoreType.DMA for _ in range(NBUF)],
        ),
    )
    def ksc(src_hbm, dst_hbm, z_hbm, *args, acc, sidx, didx,
            rowbufs, gsems, ssems):
        tabs = args[:C]
        out_hbm = args[C]
        core = lax.axis_index("c")
        sub = lax.axis_index("s")
        start = jnp.where(core == 0, sub * cpw0, NS * cpw0 + sub * cpw1)
        nb = jnp.where(core == 0, NB0, NB1)

        for c in range(C):
            table = tabs[c]
            if c == 0:
                pltpu.sync_copy(z_hbm, acc.at[pl.ds(sub * TPS, TPS)])
            plsc.subcore_barrier()

            @pl.loop(0, nb)
            def _blocks(blk):
                ib_off = (start + blk * IB) * CHUNK
                pltpu.sync_copy(
                    src_hbm.at[pl.ds(ib_off, IB * CHUNK)], sidx)
                pltpu.sync_copy(
                    dst_hbm.at[pl.ds(ib_off, IB * CHUNK)], didx)
                for b in range(NBUF):
                    pltpu.async_copy(
                        table.at[sidx.at[pl.ds(b * GRP, GRP)]],
                        rowbufs[b], gsems[b])

                @pl.loop(0, NG, step=NBUF)
                def _chunks(i0):
                    for b in range(NBUF):
                        j = i0 + b
                        pltpu.make_async_copy(
                            table.at[sidx.at[pl.ds(j * GRP, GRP)]],
                            rowbufs[b], gsems[b]).wait()
                        pltpu.async_copy(
                            rowbufs[b],
                            acc.at[didx.at[pl.ds(j * GRP, GRP)]],
                            ssems[b], add=True)

                        @pl.when(j + NBUF < NG)
                        def _():
                            pltpu.make_async_copy(
                                rowbufs[b],
                                acc.at[didx.at[pl.ds(j * GRP, GRP)]],
                                ssems[b]).wait()
                            pltpu.async_copy(
                                table.at[sidx.at[pl.ds((j + NBUF) * GRP, GRP)]],
                                rowbufs[b], gsems[b])

                for b in range(NBUF):
                    j = NG - NBUF + b
                    pltpu.make_async_copy(
                        rowbufs[j % NBUF],
                        acc.at[didx.at[pl.ds(j * GRP, GRP)]],
                        ssems[j % NBUF]).wait()

            plsc.subcore_barrier()
            base = (core * C + c) * NP + sub * TPS
            pltpu.sync_copy(acc.at[pl.ds(sub * TPS, TPS)],
                            out_hbm.at[pl.ds(base, TPS)])

    out = ksc(src_flat, dst_flat, ztab, *tables)
    return out.reshape(NC, C, NP, 32)


def _tc_layer1(x, SEL, W1bd, b1p):
    n, k = x.shape
    bn = 512
    ng = _round_up(n, bn) // bn
    rp = ng * bn // 4

    def body(x_ref, s_ref, w_ref, b_ref, *out_refs):
        xc = jnp.clip(x_ref[...], -1.8, 1.8)
        xq = jnp.concatenate(
            [jnp.dot(s_ref[a], xc, preferred_element_type=F32)
             for a in range(4)], axis=1)
        h = jnp.dot(xq, w_ref[...], preferred_element_type=F32) + b_ref[...]
        for c in range(4):
            out_refs[c][...] = h[:, 128 * c:128 * (c + 1)]

    return pl.pallas_call(
        body,
        grid=(ng,),
        in_specs=[
            pl.BlockSpec((bn, k), lambda i: (i, 0)),
            pl.BlockSpec((4, bn // 4, bn), lambda i: (0, 0, 0)),
            pl.BlockSpec((4 * k, 512), lambda i: (0, 0)),
            pl.BlockSpec((1, 512), lambda i: (0, 0)),
        ],
        out_specs=[pl.BlockSpec((bn // 4, 128), lambda i: (i, 0))] * 4,
        out_shape=[jax.ShapeDtypeStruct((rp, 128), F32)] * 4,
    )(x, SEL, W1bd, b1p.reshape(1, 512))


def _tc_mid(p, Wbd, bp, c_out):
    rp = p.shape[2]
    bn = 128
    grid = (rp // bn,)
    cols = 128 * c_out

    def body(p_ref, w_ref, b_ref, *out_refs):
        s = [p_ref[0, c] + p_ref[1, c] for c in range(4)]
        h = jnp.concatenate(
            [jnp.maximum(s[c] - s[c - 1] if c else s[0], 0.0)
             for c in range(4)], axis=1)
        r = jnp.dot(h, w_ref[...], preferred_element_type=F32) + b_ref[...]
        for c in range(c_out):
            out_refs[c][...] = r[:, 128 * c:128 * (c + 1)]

    return pl.pallas_call(
        body,
        grid=grid,
        in_specs=[
            pl.BlockSpec((2, 4, bn, 128), lambda i: (0, 0, i, 0)),
            pl.BlockSpec((512, cols), lambda i: (0, 0)),
            pl.BlockSpec((1, cols), lambda i: (0, 0)),
        ],
        out_specs=[pl.BlockSpec((bn, 128), lambda i: (i, 0))] * c_out,
        out_shape=[jax.ShapeDtypeStruct((rp, 128), F32)] * c_out,
    )(p, Wbd, bp.reshape(1, -1))


def _tc_final(p):
    rp = p.shape[2]
    bn = 128
    grid = (rp // bn,)

    def body(p_ref, out_ref):
        out_ref[...] = p_ref[0, 0] + p_ref[1, 0]

    return pl.pallas_call(
        body,
        grid=grid,
        in_specs=[pl.BlockSpec((2, 1, bn, 128), lambda i: (0, 0, i, 0))],
        out_specs=pl.BlockSpec((bn, 128), lambda i: (i, 0)),
        out_shape=jax.ShapeDtypeStruct((rp, 128), F32),
    )(p)


def kernel(x, adj, W1, b1, W2, b2, W3, b3):
    n = x.shape[0]
    e = adj.shape[1]
    NP = _round_up(n + 1, 512)

    cpw = -(-e // (CHUNK * NC * NS))
    ep = cpw * CHUNK * NC * NS
    src_flat = jnp.concatenate(
        [adj[0], jnp.zeros((ep - e,), jnp.int32)])
    dst_flat = jnp.concatenate(
        [adj[1], jnp.full((ep - e,), n, jnp.int32)])
    split = (224, 168)
    assert NS * sum(split) == ep // CHUNK
    ztab = jnp.zeros((NP // NS, 32), F32)

    W1r = W1.reshape(W1.shape[0], 4, 32)
    K = W1.shape[0]
    W1bd = (jnp.eye(4, dtype=F32)[:, None, None, :, None]
            * W1r[None, :, :, None, :]).reshape(4 * K, 512)
    b1p = jnp.tile(b1.reshape(4, 1, 32), (1, 4, 1)).reshape(512)

    W2r = W2.reshape(4, 32, 4, 32)
    W2bd = (jnp.eye(4, dtype=F32)[None, :, None, None, :, None]
            * W2r[:, None, :, :, None, :]).reshape(512, 512)
    b2p = jnp.tile(b2.reshape(4, 1, 32), (1, 4, 1)).reshape(512)

    W3p = jnp.pad(W3, ((0, 0), (0, 32 - W3.shape[1])))
    W3r = W3p.reshape(4, 32, 32)
    W3bd = (jnp.eye(4, dtype=F32)[None, :, None, :, None]
            * W3r[:, None, :, None, :]).reshape(512, 128)
    b3p = jnp.tile(jnp.pad(b3, (0, 32 - b3.shape[0])).reshape(1, 32),
                   (4, 1)).reshape(128)

    SEL = jnp.eye(512, dtype=F32).reshape(128, 4, 512).transpose(1, 0, 2)

    g1 = _tc_layer1(x, SEL, W1bd, b1p)
    p1 = _sc_segsum(src_flat, dst_flat, ztab,
                    [t.reshape(-1, 32) for t in g1], n, split)
    g2 = _tc_mid(p1.reshape(NC, 4, NP // 4, 128), W2bd, b2p, 4)
    p2 = _sc_segsum(src_flat, dst_flat, ztab,
                    [t.reshape(-1, 32) for t in g2], n, split)
    g3 = _tc_mid(p2.reshape(NC, 4, NP // 4, 128), W3bd, b3p, 1)
    p3 = _sc_segsum(src_flat, dst_flat, ztab,
                    [t.reshape(-1, 32) for t in g3], n, split)
    out = _tc_final(p3.reshape(NC, 1, NP // 4, 128))
    return out.reshape(NP, 32)[:n, :18]


# speedup vs baseline: 1.2005x; 1.0597x over previous
import functools

import jax
import jax.numpy as jnp
from jax import lax
from jax.experimental import pallas as pl
from jax.experimental.pallas import tpu as pltpu
from jax.experimental.pallas import tpu_sc as plsc

F32 = jnp.float32

NC = 2
NS = 16
CHUNK = 128
NBUF = 4


def _round_up(a, b):
    return (a + b - 1) // b * b


def _sc_segsum(src_flat, dst_flat, ztab, tables, n_nodes, split):
    C = len(tables)
    n_chunks = src_flat.shape[0] // CHUNK
    cpw0, cpw1 = split
    assert NS * (cpw0 + cpw1) == n_chunks
    IB = 28
    GRP = CHUNK
    NG = IB * CHUNK // GRP
    NB0, NB1 = cpw0 // IB, cpw1 // IB
    NP = _round_up(n_nodes + 1, 512)
    TPS = NP // NS
    mesh = plsc.VectorSubcoreMesh(core_axis_name="c", subcore_axis_name="s")

    @functools.partial(
        pl.kernel,
        out_type=jax.ShapeDtypeStruct((NC * C * NP, 32), F32),
        mesh=mesh,
        compiler_params=pltpu.CompilerParams(use_tc_tiling_on_sc=False),
        scratch_types=dict(
            acc=pltpu.VMEM_SHARED((NP, 32), F32),
            sidx=pltpu.VMEM((IB * CHUNK,), jnp.int32),
            didx=pltpu.VMEM((IB * CHUNK,), jnp.int32),
            rowbufs=[pltpu.VMEM((GRP, 32), F32) for _ in range(NBUF)],
            gsems=[pltpu.SemaphoreType.DMA for _ in range(NBUF)],
            ssems=[pltpu.SemaphoreType.DMA for _ in range(NBUF)],
        ),
    )
    def ksc(src_hbm, dst_hbm, z_hbm, *args, acc, sidx, didx,
            rowbufs, gsems, ssems):
        tabs = args[:C]
        out_hbm = args[C]
        core = lax.axis_index("c")
        sub = lax.axis_index("s")
        start = jnp.where(core == 0, sub * cpw0, NS * cpw0 + sub * cpw1)
        nb = jnp.where(core == 0, NB0, NB1)

        for c in range(C):
            table = tabs[c]
            if c == 0:
                pltpu.sync_copy(z_hbm, acc.at[pl.ds(sub * TPS, TPS)])
            plsc.subcore_barrier()

            @pl.loop(0, nb)
            def _blocks(blk):
                ib_off = (start + blk * IB) * CHUNK
                pltpu.sync_copy(
                    src_hbm.at[pl.ds(ib_off, IB * CHUNK)], sidx)
                pltpu.sync_copy(
                    dst_hbm.at[pl.ds(ib_off, IB * CHUNK)], didx)
                for b in range(NBUF):
                    pltpu.async_copy(
                        table.at[sidx.at[pl.ds(b * GRP, GRP)]],
                        rowbufs[b], gsems[b])

                @pl.loop(0, NG, step=NBUF)
                def _chunks(i0):
                    for b in range(NBUF):
                        j = i0 + b
                        pltpu.make_async_copy(
                            table.at[sidx.at[pl.ds(j * GRP, GRP)]],
                            rowbufs[b], gsems[b]).wait()
                        pltpu.async_copy(
                            rowbufs[b],
                            acc.at[didx.at[pl.ds(j * GRP, GRP)]],
                            ssems[b], add=True)

                        @pl.when(j + NBUF < NG)
                        def _():
                            pltpu.make_async_copy(
                                rowbufs[b],
                                acc.at[didx.at[pl.ds(j * GRP, GRP)]],
                                ssems[b]).wait()
                            pltpu.async_copy(
                                table.at[sidx.at[pl.ds((j + NBUF) * GRP, GRP)]],
                                rowbufs[b], gsems[b])

                for b in range(NBUF):
                    j = NG - NBUF + b
                    pltpu.make_async_copy(
                        rowbufs[j % NBUF],
                        acc.at[didx.at[pl.ds(j * GRP, GRP)]],
                        ssems[j % NBUF]).wait()

            plsc.subcore_barrier()
            base = (core * C + c) * NP + sub * TPS
            pltpu.sync_copy(acc.at[pl.ds(sub * TPS, TPS)],
                            out_hbm.at[pl.ds(base, TPS)])

    out = ksc(src_flat, dst_flat, ztab, *tables)
    return out.reshape(NC, C, NP, 32)


def _tc_layer1(x, SEL, W1bd, b1p):
    n, k = x.shape
    bn = 512
    ng = _round_up(n, bn) // bn
    rp = ng * bn // 4

    def body(x_ref, s_ref, w_ref, b_ref, *out_refs):
        xc = jnp.clip(x_ref[...], -1.8, 1.8)
        xq = jnp.concatenate(
            [jnp.dot(s_ref[a], xc, preferred_element_type=F32)
             for a in range(4)], axis=1)
        h = jnp.dot(xq, w_ref[...], preferred_element_type=F32) + b_ref[...]
        for c in range(4):
            out_refs[c][...] = h[:, 128 * c:128 * (c + 1)]

    return pl.pallas_call(
        body,
        grid=(ng,),
        in_specs=[
            pl.BlockSpec((bn, k), lambda i: (i, 0)),
            pl.BlockSpec((4, bn // 4, bn), lambda i: (0, 0, 0)),
            pl.BlockSpec((4 * k, 512), lambda i: (0, 0)),
            pl.BlockSpec((1, 512), lambda i: (0, 0)),
        ],
        out_specs=[pl.BlockSpec((bn // 4, 128), lambda i: (i, 0))] * 4,
        out_shape=[jax.ShapeDtypeStruct((rp, 128), F32)] * 4,
    )(x, SEL, W1bd, b1p.reshape(1, 512))


def _tc_mid(p, Wbd, bp, c_out):
    rp = p.shape[2]
    bn = 256
    grid = (rp // bn,)
    cols = 128 * c_out

    def body(p_ref, w_ref, b_ref, *out_refs):
        s = [p_ref[0, c] + p_ref[1, c] for c in range(4)]
        h = jnp.concatenate(
            [jnp.maximum(s[c] - s[c - 1] if c else s[0], 0.0)
             for c in range(4)], axis=1)
        r = jnp.dot(h, w_ref[...], preferred_element_type=F32) + b_ref[...]
        for c in range(c_out):
            out_refs[c][...] = r[:, 128 * c:128 * (c + 1)]

    return pl.pallas_call(
        body,
        grid=grid,
        in_specs=[
            pl.BlockSpec((2, 4, bn, 128), lambda i: (0, 0, i, 0)),
            pl.BlockSpec((512, cols), lambda i: (0, 0)),
            pl.BlockSpec((1, cols), lambda i: (0, 0)),
        ],
        out_specs=[pl.BlockSpec((bn, 128), lambda i: (i, 0))] * c_out,
        out_shape=[jax.ShapeDtypeStruct((rp, 128), F32)] * c_out,
    )(p, Wbd, bp.reshape(1, -1))


def _tc_final(p):
    rp = p.shape[2]
    bn = 256
    grid = (rp // bn,)

    def body(p_ref, out_ref):
        out_ref[...] = p_ref[0, 0] + p_ref[1, 0]

    return pl.pallas_call(
        body,
        grid=grid,
        in_specs=[pl.BlockSpec((2, 1, bn, 128), lambda i: (0, 0, i, 0))],
        out_specs=pl.BlockSpec((bn, 128), lambda i: (i, 0)),
        out_shape=jax.ShapeDtypeStruct((rp, 128), F32),
    )(p)


def kernel(x, adj, W1, b1, W2, b2, W3, b3):
    n = x.shape[0]
    e = adj.shape[1]
    NP = _round_up(n + 1, 512)

    cpw = -(-e // (CHUNK * NC * NS))
    ep = cpw * CHUNK * NC * NS
    src_flat = jnp.concatenate(
        [adj[0], jnp.zeros((ep - e,), jnp.int32)])
    dst_flat = jnp.concatenate(
        [adj[1], jnp.full((ep - e,), n, jnp.int32)])
    split = (224, 168)
    assert NS * sum(split) == ep // CHUNK
    ztab = jnp.zeros((NP // NS, 32), F32)

    W1r = W1.reshape(W1.shape[0], 4, 32)
    K = W1.shape[0]
    W1bd = (jnp.eye(4, dtype=F32)[:, None, None, :, None]
            * W1r[None, :, :, None, :]).reshape(4 * K, 512)
    b1p = jnp.tile(b1.reshape(4, 1, 32), (1, 4, 1)).reshape(512)

    W2r = W2.reshape(4, 32, 4, 32)
    W2bd = (jnp.eye(4, dtype=F32)[None, :, None, None, :, None]
            * W2r[:, None, :, :, None, :]).reshape(512, 512)
    b2p = jnp.tile(b2.reshape(4, 1, 32), (1, 4, 1)).reshape(512)

    W3p = jnp.pad(W3, ((0, 0), (0, 32 - W3.shape[1])))
    W3r = W3p.reshape(4, 32, 32)
    W3bd = (jnp.eye(4, dtype=F32)[None, :, None, :, None]
            * W3r[:, None, :, None, :]).reshape(512, 128)
    b3p = jnp.tile(jnp.pad(b3, (0, 32 - b3.shape[0])).reshape(1, 32),
                   (4, 1)).reshape(128)

    SEL = jnp.eye(512, dtype=F32).reshape(128, 4, 512).transpose(1, 0, 2)

    g1 = _tc_layer1(x, SEL, W1bd, b1p)
    p1 = _sc_segsum(src_flat, dst_flat, ztab,
                    [t.reshape(-1, 32) for t in g1], n, split)
    g2 = _tc_mid(p1.reshape(NC, 4, NP // 4, 128), W2bd, b2p, 4)
    p2 = _sc_segsum(src_flat, dst_flat, ztab,
                    [t.reshape(-1, 32) for t in g2], n, split)
    g3 = _tc_mid(p2.reshape(NC, 4, NP // 4, 128), W3bd, b3p, 1)
    p3 = _sc_segsum(src_flat, dst_flat, ztab,
                    [t.reshape(-1, 32) for t in g3], n, split)
    out = _tc_final(p3.reshape(NC, 1, NP // 4, 128))
    return out.reshape(NP, 32)[:n, :18]


# speedup vs baseline: 1.2328x; 1.0269x over previous
import functools

import jax
import jax.numpy as jnp
from jax import lax
from jax.experimental import pallas as pl
from jax.experimental.pallas import tpu as pltpu
from jax.experimental.pallas import tpu_sc as plsc

F32 = jnp.float32

NC = 2
NS = 16
CHUNK = 128
NBUF = 4


def _round_up(a, b):
    return (a + b - 1) // b * b


def _sc_segsum(src_flat, dst_flat, ztab, tables, n_nodes, split):
    C = len(tables)
    n_chunks = src_flat.shape[0] // CHUNK
    cpw0, cpw1 = split
    assert NS * (cpw0 + cpw1) == n_chunks
    IB = 28
    GRP = CHUNK
    NG = IB * CHUNK // GRP
    NB0, NB1 = cpw0 // IB, cpw1 // IB
    NP = _round_up(n_nodes + 1, 512)
    TPS = NP // NS
    mesh = plsc.VectorSubcoreMesh(core_axis_name="c", subcore_axis_name="s")

    @functools.partial(
        pl.kernel,
        out_type=jax.ShapeDtypeStruct((NC * C * NP, 32), F32),
        mesh=mesh,
        compiler_params=pltpu.CompilerParams(use_tc_tiling_on_sc=False),
        scratch_types=dict(
            acc=pltpu.VMEM_SHARED((NP, 32), F32),
            sidx=pltpu.VMEM((IB * CHUNK,), jnp.int32),
            didx=pltpu.VMEM((IB * CHUNK,), jnp.int32),
            rowbufs=[pltpu.VMEM((GRP, 32), F32) for _ in range(NBUF)],
            gsems=[pltpu.SemaphoreType.DMA for _ in range(NBUF)],
            ssems=[pltpu.SemaphoreType.DMA for _ in range(NBUF)],
        ),
    )
    def ksc(src_hbm, dst_hbm, z_hbm, *args, acc, sidx, didx,
            rowbufs, gsems, ssems):
        tabs = args[:C]
        out_hbm = args[C]
        core = lax.axis_index("c")
        sub = lax.axis_index("s")
        start = jnp.where(core == 0, sub * cpw0, NS * cpw0 + sub * cpw1)
        nb = jnp.where(core == 0, NB0, NB1)

        for c in range(C):
            table = tabs[c]
            if c == 0:
                pltpu.sync_copy(z_hbm, acc.at[pl.ds(sub * TPS, TPS)])
            plsc.subcore_barrier()

            @pl.loop(0, nb)
            def _blocks(blk):
                ib_off = (start + blk * IB) * CHUNK
                pltpu.sync_copy(
                    src_hbm.at[pl.ds(ib_off, IB * CHUNK)], sidx)
                pltpu.sync_copy(
                    dst_hbm.at[pl.ds(ib_off, IB * CHUNK)], didx)
                for b in range(NBUF):
                    pltpu.async_copy(
                        table.at[sidx.at[pl.ds(b * GRP, GRP)]],
                        rowbufs[b], gsems[b])

                @pl.loop(0, NG, step=NBUF)
                def _chunks(i0):
                    for b in range(NBUF):
                        j = i0 + b
                        pltpu.make_async_copy(
                            table.at[sidx.at[pl.ds(j * GRP, GRP)]],
                            rowbufs[b], gsems[b]).wait()
                        pltpu.async_copy(
                            rowbufs[b],
                            acc.at[didx.at[pl.ds(j * GRP, GRP)]],
                            ssems[b], add=True)

                        @pl.when(j + NBUF < NG)
                        def _():
                            pltpu.make_async_copy(
                                rowbufs[b],
                                acc.at[didx.at[pl.ds(j * GRP, GRP)]],
                                ssems[b]).wait()
                            pltpu.async_copy(
                                table.at[sidx.at[pl.ds((j + NBUF) * GRP, GRP)]],
                                rowbufs[b], gsems[b])

                for b in range(NBUF):
                    j = NG - NBUF + b
                    pltpu.make_async_copy(
                        rowbufs[j % NBUF],
                        acc.at[didx.at[pl.ds(j * GRP, GRP)]],
                        ssems[j % NBUF]).wait()

            plsc.subcore_barrier()
            base = (core * C + c) * NP + sub * TPS
            pltpu.sync_copy(acc.at[pl.ds(sub * TPS, TPS)],
                            out_hbm.at[pl.ds(base, TPS)])

    out = ksc(src_flat, dst_flat, ztab, *tables)
    return out.reshape(NC, C, NP, 32)


def _tc_layer1(x, SEL, W1bd, b1p):
    n, k = x.shape
    bn = 512
    ng = _round_up(n, bn) // bn
    rp = ng * bn // 4

    def body(x_ref, s_ref, w_ref, b_ref, *out_refs):
        xc = jnp.clip(x_ref[...], -1.8, 1.8)
        xq = jnp.concatenate(
            [jnp.dot(s_ref[a], xc, preferred_element_type=F32)
             for a in range(4)], axis=1)
        h = jnp.dot(xq, w_ref[...], preferred_element_type=F32) + b_ref[...]
        for c in range(4):
            out_refs[c][...] = h[:, 128 * c:128 * (c + 1)]

    return pl.pallas_call(
        body,
        grid=(ng,),
        in_specs=[
            pl.BlockSpec((bn, k), lambda i: (i, 0)),
            pl.BlockSpec((4, bn // 4, bn), lambda i: (0, 0, 0)),
            pl.BlockSpec((4 * k, 512), lambda i: (0, 0)),
            pl.BlockSpec((1, 512), lambda i: (0, 0)),
        ],
        out_specs=[pl.BlockSpec((bn // 4, 128), lambda i: (i, 0))] * 4,
        out_shape=[jax.ShapeDtypeStruct((rp, 128), F32)] * 4,
    )(x, SEL, W1bd, b1p.reshape(1, 512))


def _tc_mid(p, Wbd, bp, c_out):
    rp = p.shape[2]
    bn = 448
    grid = (rp // bn,)
    cols = 128 * c_out

    def body(p_ref, w_ref, b_ref, *out_refs):
        s = [p_ref[0, c] + p_ref[1, c] for c in range(4)]
        h = jnp.concatenate(
            [jnp.maximum(s[c] - s[c - 1] if c else s[0], 0.0)
             for c in range(4)], axis=1)
        r = jnp.dot(h, w_ref[...], preferred_element_type=F32) + b_ref[...]
        for c in range(c_out):
            out_refs[c][...] = r[:, 128 * c:128 * (c + 1)]

    return pl.pallas_call(
        body,
        grid=grid,
        in_specs=[
            pl.BlockSpec((2, 4, bn, 128), lambda i: (0, 0, i, 0)),
            pl.BlockSpec((512, cols), lambda i: (0, 0)),
            pl.BlockSpec((1, cols), lambda i: (0, 0)),
        ],
        out_specs=[pl.BlockSpec((bn, 128), lambda i: (i, 0))] * c_out,
        out_shape=[jax.ShapeDtypeStruct((rp, 128), F32)] * c_out,
    )(p, Wbd, bp.reshape(1, -1))


def _tc_final(p):
    rp = p.shape[2]
    bn = 448
    grid = (rp // bn,)

    def body(p_ref, out_ref):
        out_ref[...] = p_ref[0, 0] + p_ref[1, 0]

    return pl.pallas_call(
        body,
        grid=grid,
        in_specs=[pl.BlockSpec((2, 1, bn, 128), lambda i: (0, 0, i, 0))],
        out_specs=pl.BlockSpec((bn, 128), lambda i: (i, 0)),
        out_shape=jax.ShapeDtypeStruct((rp, 128), F32),
    )(p)


def kernel(x, adj, W1, b1, W2, b2, W3, b3):
    n = x.shape[0]
    e = adj.shape[1]
    NP = _round_up(n + 1, 512)

    cpw = -(-e // (CHUNK * NC * NS))
    ep = cpw * CHUNK * NC * NS
    src_flat = jnp.concatenate(
        [adj[0], jnp.zeros((ep - e,), jnp.int32)])
    dst_flat = jnp.concatenate(
        [adj[1], jnp.full((ep - e,), n, jnp.int32)])
    split = (224, 168)
    assert NS * sum(split) == ep // CHUNK
    ztab = jnp.zeros((NP // NS, 32), F32)

    W1r = W1.reshape(W1.shape[0], 4, 32)
    K = W1.shape[0]
    W1bd = (jnp.eye(4, dtype=F32)[:, None, None, :, None]
            * W1r[None, :, :, None, :]).reshape(4 * K, 512)
    b1p = jnp.tile(b1.reshape(4, 1, 32), (1, 4, 1)).reshape(512)

    W2r = W2.reshape(4, 32, 4, 32)
    W2bd = (jnp.eye(4, dtype=F32)[None, :, None, None, :, None]
            * W2r[:, None, :, :, None, :]).reshape(512, 512)
    b2p = jnp.tile(b2.reshape(4, 1, 32), (1, 4, 1)).reshape(512)

    W3p = jnp.pad(W3, ((0, 0), (0, 32 - W3.shape[1])))
    W3r = W3p.reshape(4, 32, 32)
    W3bd = (jnp.eye(4, dtype=F32)[None, :, None, :, None]
            * W3r[:, None, :, None, :]).reshape(512, 128)
    b3p = jnp.tile(jnp.pad(b3, (0, 32 - b3.shape[0])).reshape(1, 32),
                   (4, 1)).reshape(128)

    SEL = jnp.eye(512, dtype=F32).reshape(128, 4, 512).transpose(1, 0, 2)

    g1 = _tc_layer1(x, SEL, W1bd, b1p)
    p1 = _sc_segsum(src_flat, dst_flat, ztab,
                    [t.reshape(-1, 32) for t in g1], n, split)
    g2 = _tc_mid(p1.reshape(NC, 4, NP // 4, 128), W2bd, b2p, 4)
    p2 = _sc_segsum(src_flat, dst_flat, ztab,
                    [t.reshape(-1, 32) for t in g2], n, split)
    g3 = _tc_mid(p2.reshape(NC, 4, NP // 4, 128), W3bd, b3p, 1)
    p3 = _sc_segsum(src_flat, dst_flat, ztab,
                    [t.reshape(-1, 32) for t in g3], n, split)
    out = _tc_final(p3.reshape(NC, 1, NP // 4, 128))
    return out.reshape(NP, 32)[:n, :18]
